# baseline (speedup 1.0000x reference)
import functools

import jax
import jax.numpy as jnp
from jax import lax
from jax.experimental import pallas as pl
from jax.experimental.pallas import tpu as pltpu
from jax.experimental.pallas import tpu_sc as plsc

B, N, C, H = 2, 4096, 768, 12
DH = C // H
K_SEL = 409
K_PAD = 512
NB = 8
BN = N // NB
ONE_BITS = 0x3F800000


def _topk_body(row_ref, idx_ref):
    b = pl.program_id(0)
    row = row_ref[0]
    bits = jax.lax.bitcast_convert_type(row, jnp.int32)
    pos = jax.lax.broadcasted_iota(jnp.int32, (1, N), 1)
    bits = jnp.where(pos == 0, -1, bits)

    def bisect(_, carry):
        lo, hi = carry
        mid = (lo + hi) // 2
        cnt = jnp.sum((bits > mid).astype(jnp.int32))
        big = cnt >= K_SEL
        return jnp.where(big, mid, lo), jnp.where(big, hi, mid)

    lo, hi = lax.fori_loop(0, 31, bisect, (jnp.int32(-1), jnp.int32(ONE_BITS)))
    thr = hi

    gt = (bits > thr).astype(jnp.int32)
    eq = (bits == thr).astype(jnp.int32)
    n_gt = jnp.sum(gt)

    def cumsum_lanes(v):
        acc = v
        for s in (1, 2, 4, 8, 16, 32, 64, 128, 256, 512, 1024, 2048):
            shifted = jnp.concatenate(
                [jnp.zeros((1, s), jnp.int32), acc[:, : N - s]], axis=1)
            acc = acc + shifted
        return acc

    cgt = cumsum_lanes(gt)
    ceq = cumsum_lanes(eq)
    rank = jnp.where(gt == 1, cgt - 1,
                     jnp.where(eq == 1, n_gt + ceq - 1, jnp.int32(N)))

    jcol = jax.lax.broadcasted_iota(jnp.int32, (K_PAD, 1), 0)
    onehot = rank == jcol
    idx_j = jnp.sum(jnp.where(onehot, pos, 0), axis=1)
    idx0 = jnp.sum(jnp.where(rank == 0, pos, 0))
    jvec = jax.lax.iota(jnp.int32, K_PAD)
    idx_flat = jnp.where(jvec < K_SEL, idx_j, idx0) + b * N
    idx_ref[...] = idx_flat.reshape(1, 1, K_PAD)


def _topk_indices(row0):
    return pl.pallas_call(
        _topk_body,
        grid=(B,),
        in_specs=[pl.BlockSpec((1, 1, N), lambda b: (b, 0, 0))],
        out_specs=pl.BlockSpec((1, 1, K_PAD), lambda b: (b, 0, 0)),
        out_shape=jax.ShapeDtypeStruct((B, 1, K_PAD), jnp.int32),
    )(row0)


_ROWS_PER_W = (B * K_PAD) // 32


@functools.cache
def _sc_gather():
    mesh = plsc.VectorSubcoreMesh(core_axis_name="c", subcore_axis_name="s")

    @functools.partial(
        pl.kernel,
        out_type=jax.ShapeDtypeStruct((B * K_PAD, C), jnp.float32),
        mesh=mesh,
        scratch_types=[
            pltpu.VMEM((_ROWS_PER_W,), jnp.int32),
            pltpu.VMEM((_ROWS_PER_W, C), jnp.float32),
            pltpu.SemaphoreType.DMA,
        ],
    )
    def gather(x_hbm, idx_hbm, out_hbm, idx_v, rows_v, sem):
        wid = lax.axis_index("s") * 2 + lax.axis_index("c")
        base = wid * _ROWS_PER_W
        pltpu.sync_copy(idx_hbm.at[pl.ds(base, _ROWS_PER_W)], idx_v)
        pltpu.async_copy(x_hbm.at[idx_v], rows_v, sem).wait()
        pltpu.sync_copy(rows_v, out_hbm.at[pl.ds(base, _ROWS_PER_W)])

    return gather


def _attn_body(selx_ref, x_ref, wqt_ref, bq_ref, wkvt_ref, bkv_ref,
               wpt_ref, bp_ref, out_ref, q_s, m_s, l_s, acc_s):
    n = pl.program_id(1)
    scale = DH ** -0.5

    @pl.when(n == 0)
    def _init():
        selx = selx_ref[0].astype(jnp.bfloat16)
        q_s[...] = (jnp.dot(selx, wqt_ref[...].astype(jnp.bfloat16),
                            preferred_element_type=jnp.float32)
                    + bq_ref[...]).astype(jnp.bfloat16)
        m_s[...] = jnp.full((K_PAD, 128), -1e30, jnp.float32)
        l_s[...] = jnp.zeros((K_PAD, 128), jnp.float32)
        acc_s[...] = jnp.zeros((K_PAD, C), jnp.float32)

    xb = x_ref[0].astype(jnp.bfloat16)
    kv = (jnp.dot(xb, wkvt_ref[...].astype(jnp.bfloat16),
                  preferred_element_type=jnp.float32)
          + bkv_ref[...]).astype(jnp.bfloat16)

    for h in range(H):
        sl = slice(h * DH, (h + 1) * DH)
        vsl = slice(C + h * DH, C + (h + 1) * DH)
        qh = q_s[:, sl]
        kh = kv[:, sl]
        vh = kv[:, vsl]
        s = lax.dot_general(qh, kh, (((1,), (1,)), ((), ())),
                            preferred_element_type=jnp.float32) * scale
        m_old = m_s[:, h:h + 1]
        m_new = jnp.maximum(m_old, jnp.max(s, axis=1, keepdims=True))
        p = jnp.exp(s - m_new)
        corr = jnp.exp(m_old - m_new)
        l_s[:, h:h + 1] = l_s[:, h:h + 1] * corr + jnp.sum(p, axis=1,
                                                           keepdims=True)
        acc_s[:, sl] = acc_s[:, sl] * corr + jnp.dot(
            p.astype(jnp.bfloat16), vh, preferred_element_type=jnp.float32)
        m_s[:, h:h + 1] = m_new

    @pl.when(n == NB - 1)
    def _final():
        ctx = jnp.concatenate(
            [acc_s[:, h * DH:(h + 1) * DH] / l_s[:, h:h + 1]
             for h in range(H)], axis=1).astype(jnp.bfloat16)
        out_ref[0] = jnp.dot(ctx, wpt_ref[...].astype(jnp.bfloat16),
                             preferred_element_type=jnp.float32) + bp_ref[...]


def _attention(selx, x, wqt, bq2, wkvt, bkv2, wpt, bp2):
    return pl.pallas_call(
        _attn_body,
        grid=(B, NB),
        in_specs=[
            pl.BlockSpec((1, K_PAD, C), lambda b, n: (b, 0, 0)),
            pl.BlockSpec((1, BN, C), lambda b, n: (b, n, 0)),
            pl.BlockSpec((C, C), lambda b, n: (0, 0)),
            pl.BlockSpec((1, C), lambda b, n: (0, 0)),
            pl.BlockSpec((C, 2 * C), lambda b, n: (0, 0)),
            pl.BlockSpec((1, 2 * C), lambda b, n: (0, 0)),
            pl.BlockSpec((C, C), lambda b, n: (0, 0)),
            pl.BlockSpec((1, C), lambda b, n: (0, 0)),
        ],
        out_specs=pl.BlockSpec((1, K_PAD, C), lambda b, n: (b, 0, 0)),
        out_shape=jax.ShapeDtypeStruct((B, K_PAD, C), jnp.float32),
        scratch_shapes=[
            pltpu.VMEM((K_PAD, C), jnp.bfloat16),
            pltpu.VMEM((K_PAD, 128), jnp.float32),
            pltpu.VMEM((K_PAD, 128), jnp.float32),
            pltpu.VMEM((K_PAD, C), jnp.float32),
        ],
        compiler_params=pltpu.CompilerParams(
            dimension_semantics=("arbitrary", "arbitrary"),
        ),
    )(selx, x, wqt, bq2, wkvt, bkv2, wpt, bp2)


_COPY_ROWS = N // 16
_COPY_CHUNK = 128
_SCAT_PER_W = K_PAD // 16


@functools.cache
def _sc_scatter():
    mesh = plsc.VectorSubcoreMesh(core_axis_name="c", subcore_axis_name="s")

    @functools.partial(
        pl.kernel,
        out_type=jax.ShapeDtypeStruct((B * N, C), jnp.float32),
        mesh=mesh,
        scratch_types=[
            pltpu.VMEM((_COPY_CHUNK, C), jnp.float32),
            pltpu.VMEM((_SCAT_PER_W,), jnp.int32),
            pltpu.VMEM((_SCAT_PER_W, C), jnp.float32),
            pltpu.SemaphoreType.DMA,
        ],
    )
    def scatter(x_hbm, idx_hbm, loc_hbm, out_hbm, buf_v, idx_v, rows_v, sem):
        c = lax.axis_index("c")
        s = lax.axis_index("s")
        copy_base = c * N + s * _COPY_ROWS
        for t in range(_COPY_ROWS // _COPY_CHUNK):
            off = copy_base + t * _COPY_CHUNK
            pltpu.sync_copy(x_hbm.at[pl.ds(off, _COPY_CHUNK)], buf_v)
            pltpu.sync_copy(buf_v, out_hbm.at[pl.ds(off, _COPY_CHUNK)])
        plsc.subcore_barrier()
        scat_base = c * K_PAD + s * _SCAT_PER_W
        pltpu.sync_copy(idx_hbm.at[pl.ds(scat_base, _SCAT_PER_W)], idx_v)
        pltpu.sync_copy(loc_hbm.at[pl.ds(scat_base, _SCAT_PER_W)], rows_v)
        pltpu.async_copy(rows_v, out_hbm.at[idx_v], sem).wait()

    return scatter


def kernel(x, attention_rollout, Wq, bq, Wkv, bkv, Wp, bp):
    row0 = attention_rollout[:, 0, :].reshape(B, 1, N)
    idx_flat = _topk_indices(row0).reshape(B * K_PAD)
    x2d = x.reshape(B * N, C)
    selx = _sc_gather()(x2d, idx_flat).reshape(B, K_PAD, C)
    local_out = _attention(
        selx, x, Wq.T, bq.reshape(1, C), Wkv.T, bkv.reshape(1, 2 * C),
        Wp.T, bp.reshape(1, C))
    out = _sc_scatter()(x2d, idx_flat, local_out.reshape(B * K_PAD, C))
    return out.reshape(B, N, C)


# speedup vs baseline: 1.3154x; 1.3154x over previous
import functools

import jax
import jax.numpy as jnp
from jax import lax
from jax.experimental import pallas as pl
from jax.experimental.pallas import tpu as pltpu
from jax.experimental.pallas import tpu_sc as plsc

B, N, C, H = 2, 4096, 768, 12
DH = C // H
K_SEL = 409
K_PAD = 512
NB = 8
BN = N // NB
ONE_BITS = 0x3F800000


def _topk_body(row_ref, idx_ref):
    b = pl.program_id(0)
    row = row_ref[0]
    bits = jax.lax.bitcast_convert_type(row, jnp.int32)
    pos = jax.lax.broadcasted_iota(jnp.int32, (1, N), 1)
    bits = jnp.where(pos == 0, -1, bits)

    def bisect(_, carry):
        lo, hi = carry
        mid = (lo + hi) // 2
        cnt = jnp.sum((bits > mid).astype(jnp.int32))
        big = cnt >= K_SEL
        return jnp.where(big, mid, lo), jnp.where(big, hi, mid)

    lo, hi = lax.fori_loop(0, 31, bisect, (jnp.int32(-1), jnp.int32(ONE_BITS)))
    thr = hi

    gt = (bits > thr).astype(jnp.int32)
    eq = (bits == thr).astype(jnp.int32)
    n_gt = jnp.sum(gt)

    def cumsum_lanes(v):
        acc = v
        for s in (1, 2, 4, 8, 16, 32, 64, 128, 256, 512, 1024, 2048):
            shifted = jnp.concatenate(
                [jnp.zeros((1, s), jnp.int32), acc[:, : N - s]], axis=1)
            acc = acc + shifted
        return acc

    cgt = cumsum_lanes(gt)
    ceq = cumsum_lanes(eq)
    rank = jnp.where(gt == 1, cgt - 1,
                     jnp.where(eq == 1, n_gt + ceq - 1, jnp.int32(N)))

    jcol = jax.lax.broadcasted_iota(jnp.int32, (K_PAD, 1), 0)
    onehot = rank == jcol
    idx_j = jnp.sum(jnp.where(onehot, pos, 0), axis=1)
    idx0 = jnp.sum(jnp.where(rank == 0, pos, 0))
    jvec = jax.lax.iota(jnp.int32, K_PAD)
    idx_flat = jnp.where(jvec < K_SEL, idx_j, idx0) + b * N
    idx_ref[...] = idx_flat.reshape(1, 1, K_PAD)


def _topk_indices(row0):
    return pl.pallas_call(
        _topk_body,
        grid=(B,),
        in_specs=[pl.BlockSpec((1, 1, N), lambda b: (b, 0, 0))],
        out_specs=pl.BlockSpec((1, 1, K_PAD), lambda b: (b, 0, 0)),
        out_shape=jax.ShapeDtypeStruct((B, 1, K_PAD), jnp.int32),
    )(row0)


_ROWS_PER_W = (B * K_PAD) // 32


@functools.cache
def _sc_gather():
    mesh = plsc.VectorSubcoreMesh(core_axis_name="c", subcore_axis_name="s")

    @functools.partial(
        pl.kernel,
        out_type=jax.ShapeDtypeStruct((B * K_PAD, C), jnp.float32),
        mesh=mesh,
        scratch_types=[
            pltpu.VMEM((_ROWS_PER_W,), jnp.int32),
            pltpu.VMEM((_ROWS_PER_W, C), jnp.float32),
            pltpu.SemaphoreType.DMA,
        ],
    )
    def gather(x_hbm, idx_hbm, out_hbm, idx_v, rows_v, sem):
        wid = lax.axis_index("s") * 2 + lax.axis_index("c")
        base = wid * _ROWS_PER_W
        pltpu.sync_copy(idx_hbm.at[pl.ds(base, _ROWS_PER_W)], idx_v)
        pltpu.async_copy(x_hbm.at[idx_v], rows_v, sem).wait()
        pltpu.sync_copy(rows_v, out_hbm.at[pl.ds(base, _ROWS_PER_W)])

    return gather


def _proj_body(selx_ref, x_ref, wqt_ref, bq_ref, wkvt_ref, bkv_ref,
               kv_ref, q_ref):
    n = pl.program_id(1)

    @pl.when(n == 0)
    def _q():
        selx = selx_ref[0].astype(jnp.bfloat16)
        q = jnp.dot(selx, wqt_ref[...],
                    preferred_element_type=jnp.float32) + bq_ref[...]
        qb = q.astype(jnp.bfloat16)
        for h in range(H):
            q_ref[0, h] = qb[:, h * DH:(h + 1) * DH]

    xb = x_ref[0].astype(jnp.bfloat16)
    kv = jnp.dot(xb, wkvt_ref[...],
                 preferred_element_type=jnp.float32) + bkv_ref[...]
    kvb = kv.astype(jnp.bfloat16)
    for g in range(2 * H):
        kv_ref[0, g] = kvb[:, g * DH:(g + 1) * DH]


def _project(selx, x, wqt_b, bq2, wkvt_b, bkv2):
    return pl.pallas_call(
        _proj_body,
        grid=(B, NB),
        in_specs=[
            pl.BlockSpec((1, K_PAD, C), lambda b, n: (b, 0, 0)),
            pl.BlockSpec((1, BN, C), lambda b, n: (b, n, 0)),
            pl.BlockSpec((C, C), lambda b, n: (0, 0)),
            pl.BlockSpec((1, C), lambda b, n: (0, 0)),
            pl.BlockSpec((C, 2 * C), lambda b, n: (0, 0)),
            pl.BlockSpec((1, 2 * C), lambda b, n: (0, 0)),
        ],
        out_specs=[
            pl.BlockSpec((1, 2 * H, BN, DH), lambda b, n: (b, 0, n, 0)),
            pl.BlockSpec((1, H, K_PAD, DH), lambda b, n: (b, 0, 0, 0)),
        ],
        out_shape=[
            jax.ShapeDtypeStruct((B, 2 * H, N, DH), jnp.bfloat16),
            jax.ShapeDtypeStruct((B, H, K_PAD, DH), jnp.bfloat16),
        ],
        compiler_params=pltpu.CompilerParams(
            dimension_semantics=("arbitrary", "arbitrary"),
        ),
    )(selx, x, wqt_b, bq2, wkvt_b, bkv2)


def _head_body(q_ref, k_ref, v_ref, ctx_ref):
    scale = DH ** -0.5
    qh = q_ref[0, 0]
    kh = k_ref[0, 0]
    vh = v_ref[0, 0]
    s = lax.dot_general(qh, kh, (((1,), (1,)), ((), ())),
                        preferred_element_type=jnp.float32) * scale
    m = jnp.max(s, axis=1, keepdims=True)
    p = jnp.exp(s - m)
    l = jnp.sum(p, axis=1, keepdims=True)
    o = jnp.dot(p.astype(jnp.bfloat16), vh,
                preferred_element_type=jnp.float32)
    ctx_ref[0, 0] = o / l


def _head_attention(q, kv):
    return pl.pallas_call(
        _head_body,
        grid=(B, H),
        in_specs=[
            pl.BlockSpec((1, 1, K_PAD, DH), lambda b, h: (b, h, 0, 0)),
            pl.BlockSpec((1, 1, N, DH), lambda b, h: (b, h, 0, 0)),
            pl.BlockSpec((1, 1, N, DH), lambda b, h: (b, H + h, 0, 0)),
        ],
        out_specs=pl.BlockSpec((1, 1, K_PAD, DH), lambda b, h: (b, h, 0, 0)),
        out_shape=jax.ShapeDtypeStruct((B, H, K_PAD, DH), jnp.float32),
        compiler_params=pltpu.CompilerParams(
            dimension_semantics=("arbitrary", "arbitrary"),
        ),
    )(q, kv, kv)


def _outproj_body(ctx_ref, wpt_ref, bp_ref, out_ref):
    ctx = jnp.concatenate([ctx_ref[0, h] for h in range(H)],
                          axis=1).astype(jnp.bfloat16)
    out_ref[0] = jnp.dot(ctx, wpt_ref[...],
                         preferred_element_type=jnp.float32) + bp_ref[...]


def _outproj(ctx, wpt_b, bp2):
    return pl.pallas_call(
        _outproj_body,
        grid=(B,),
        in_specs=[
            pl.BlockSpec((1, H, K_PAD, DH), lambda b: (b, 0, 0, 0)),
            pl.BlockSpec((C, C), lambda b: (0, 0)),
            pl.BlockSpec((1, C), lambda b: (0, 0)),
        ],
        out_specs=pl.BlockSpec((1, K_PAD, C), lambda b: (b, 0, 0)),
        out_shape=jax.ShapeDtypeStruct((B, K_PAD, C), jnp.float32),
    )(ctx, wpt_b, bp2)


_COPY_ROWS = N // 16
_COPY_CHUNK = 128
_SCAT_PER_W = K_PAD // 16


@functools.cache
def _sc_scatter():
    mesh = plsc.VectorSubcoreMesh(core_axis_name="c", subcore_axis_name="s")

    @functools.partial(
        pl.kernel,
        out_type=jax.ShapeDtypeStruct((B * N, C), jnp.float32),
        mesh=mesh,
        scratch_types=[
            pltpu.VMEM((_COPY_CHUNK, C), jnp.float32),
            pltpu.VMEM((_SCAT_PER_W,), jnp.int32),
            pltpu.VMEM((_SCAT_PER_W, C), jnp.float32),
            pltpu.SemaphoreType.DMA,
        ],
    )
    def scatter(x_hbm, idx_hbm, loc_hbm, out_hbm, buf_v, idx_v, rows_v, sem):
        c = lax.axis_index("c")
        s = lax.axis_index("s")
        copy_base = c * N + s * _COPY_ROWS
        for t in range(_COPY_ROWS // _COPY_CHUNK):
            off = copy_base + t * _COPY_CHUNK
            pltpu.sync_copy(x_hbm.at[pl.ds(off, _COPY_CHUNK)], buf_v)
            pltpu.sync_copy(buf_v, out_hbm.at[pl.ds(off, _COPY_CHUNK)])
        plsc.subcore_barrier()
        scat_base = c * K_PAD + s * _SCAT_PER_W
        pltpu.sync_copy(idx_hbm.at[pl.ds(scat_base, _SCAT_PER_W)], idx_v)
        pltpu.sync_copy(loc_hbm.at[pl.ds(scat_base, _SCAT_PER_W)], rows_v)
        pltpu.async_copy(rows_v, out_hbm.at[idx_v], sem).wait()

    return scatter


def kernel(x, attention_rollout, Wq, bq, Wkv, bkv, Wp, bp):
    row0 = attention_rollout[:, 0, :].reshape(B, 1, N)
    idx_flat = _topk_indices(row0).reshape(B * K_PAD)
    x2d = x.reshape(B * N, C)
    selx = _sc_gather()(x2d, idx_flat).reshape(B, K_PAD, C)
    kv, q = _project(selx, x, Wq.T.astype(jnp.bfloat16), bq.reshape(1, C),
                     Wkv.T.astype(jnp.bfloat16), bkv.reshape(1, 2 * C))
    ctx = _head_attention(q, kv)
    local_out = _outproj(ctx, Wp.T.astype(jnp.bfloat16), bp.reshape(1, C))
    out = _sc_scatter()(x2d, idx_flat, local_out.reshape(B * K_PAD, C))
    return out.reshape(B, N, C)
